# baseline (device time: 25734 ns/iter reference)
import jax
import jax.numpy as jnp
from jax import lax
from jax.experimental import pallas as pl
from jax.experimental.pallas import tpu as pltpu

N_DEV = 4
N_STREAMS = 16
ORDER = list(range(N_STREAMS))


def kernel(x):
    m, n = x.shape
    sub = m // N_STREAMS
    q = sub // 2
    e = sub // 4

    def body(x_ref, out_ref, *scratch):
        recvs1 = scratch[0:N_STREAMS]
        recvs2 = scratch[N_STREAMS:2 * N_STREAMS]
        accs = scratch[2 * N_STREAMS:3 * N_STREAMS]
        send_sems, recv_sems = scratch[3 * N_STREAMS], scratch[3 * N_STREAMS + 1]

        my = lax.axis_index("i")
        cx = (my >> 1) & 1
        cy = (my ^ (my >> 1)) & 1
        py = my ^ 1
        px = 3 - my

        barrier_sem = pltpu.get_barrier_semaphore()
        for nbr in [py, px]:
            pl.semaphore_signal(
                barrier_sem, inc=1,
                device_id=(nbr,), device_id_type=pl.DeviceIdType.MESH,
            )
        pl.semaphore_wait(barrier_sem, 2)

        def exch(idx, src, dst, partner):
            return pltpu.make_async_remote_copy(
                src_ref=src, dst_ref=dst,
                send_sem=send_sems.at[idx], recv_sem=recv_sems.at[idx],
                device_id=(partner,), device_id_type=pl.DeviceIdType.MESH,
            )

        def geom(s):
            base = s * sub
            if s % 2 == 0:
                k1, k2 = cy, cx
                p1, p2 = py, px
            else:
                k1, k2 = cx, cy
                p1, p2 = px, py
            keep = base + k1 * q
            red = keep + k2 * e
            return base, k1, k2, p1, p2, keep, red

        G = [geom(s) for s in range(N_STREAMS)]
        sem = lambda s, st: 4 * s + st

        st1 = {}
        for s in ORDER:
            base, k1, _, p1, _, _, _ = G[s]
            r = exch(sem(s, 0), x_ref.at[pl.ds(base + (1 - k1) * q, q)],
                     recvs1[s], p1)
            r.start()
            st1[s] = r

        st2 = {}
        for s in ORDER:
            _, _, k2, _, p2, keep, _ = G[s]
            st1[s].wait_recv()
            off = (1 - k2) * e
            accs[s][...] = (
                x_ref[pl.ds(keep + off, e), :] + recvs1[s][pl.ds(off, e), :]
            )
            r = exch(sem(s, 1), accs[s], recvs2[s], p2)
            r.start()
            st2[s] = r

        st3 = {}
        for s in ORDER:
            _, _, k2, _, p2, keep, red = G[s]
            st2[s].wait_recv()
            off = k2 * e
            out_ref[pl.ds(red, e), :] = (
                x_ref[pl.ds(keep + off, e), :]
                + recvs1[s][pl.ds(off, e), :]
                + recvs2[s][...]
            )
            r = exch(sem(s, 2), out_ref.at[pl.ds(red, e)],
                     out_ref.at[pl.ds(red, e)], p2)
            r.start()
            st3[s] = r

        st4 = {}
        for s in ORDER:
            _, _, _, p1, _, keep, _ = G[s]
            st3[s].wait_recv()
            r = exch(sem(s, 3), out_ref.at[pl.ds(keep, q)],
                     out_ref.at[pl.ds(keep, q)], p1)
            r.start()
            st4[s] = r

        for s in ORDER:
            st4[s].wait_recv()
        for rs in (st1, st2, st3, st4):
            for s in ORDER:
                rs[s].wait_send()

    return pl.pallas_call(
        body,
        out_shape=jax.ShapeDtypeStruct((m, n), x.dtype),
        in_specs=[pl.BlockSpec(memory_space=pltpu.VMEM)],
        out_specs=pl.BlockSpec(memory_space=pltpu.VMEM),
        scratch_shapes=(
            [pltpu.VMEM((q, n), x.dtype) for _ in range(N_STREAMS)]
            + [pltpu.VMEM((e, n), x.dtype) for _ in range(N_STREAMS)]
            + [pltpu.VMEM((e, n), x.dtype) for _ in range(N_STREAMS)]
            + [
                pltpu.SemaphoreType.DMA((4 * N_STREAMS,)),
                pltpu.SemaphoreType.DMA((4 * N_STREAMS,)),
            ]
        ),
        compiler_params=pltpu.CompilerParams(collective_id=0),
    )(x)


# device time: 24980 ns/iter; 1.0302x vs baseline; 1.0302x over previous
import jax
import jax.numpy as jnp
from jax import lax
from jax.experimental import pallas as pl
from jax.experimental.pallas import tpu as pltpu

N_DEV = 4
N_STREAMS = 8
ORDER = list(range(N_STREAMS))


def kernel(x):
    m, n = x.shape
    sub = m // N_STREAMS
    h = sub // 2

    def body(x_ref, out_ref, *scratch):
        recvs1 = scratch[0:N_STREAMS]
        recvs2 = scratch[N_STREAMS:2 * N_STREAMS]
        accs = scratch[2 * N_STREAMS:3 * N_STREAMS]
        send_sems, recv_sems = scratch[3 * N_STREAMS], scratch[3 * N_STREAMS + 1]

        my = lax.axis_index("i")
        cx = (my >> 1) & 1
        cy = (my ^ (my >> 1)) & 1
        py = my ^ 1
        px = 3 - my

        barrier_sem = pltpu.get_barrier_semaphore()
        for nbr in [py, px]:
            pl.semaphore_signal(
                barrier_sem, inc=1,
                device_id=(nbr,), device_id_type=pl.DeviceIdType.MESH,
            )
        pl.semaphore_wait(barrier_sem, 2)

        def exch(idx, src, dst, partner):
            return pltpu.make_async_remote_copy(
                src_ref=src, dst_ref=dst,
                send_sem=send_sems.at[idx], recv_sem=recv_sems.at[idx],
                device_id=(partner,), device_id_type=pl.DeviceIdType.MESH,
            )

        def geom(s):
            base = s * sub
            if s % 2 == 0:
                k1, p1, p2 = cy, py, px
            else:
                k1, p1, p2 = cx, px, py
            keep = base + k1 * h
            other = base + (1 - k1) * h
            return k1, p1, p2, keep, other

        G = [geom(s) for s in range(N_STREAMS)]
        sem = lambda s, st: 3 * s + st

        st1 = {}
        for s in ORDER:
            _, p1, _, _, other = G[s]
            r = exch(sem(s, 0), x_ref.at[pl.ds(other, h)], recvs1[s], p1)
            r.start()
            st1[s] = r

        st2 = {}
        for s in ORDER:
            _, _, p2, keep, _ = G[s]
            st1[s].wait_recv()
            accs[s][...] = x_ref[pl.ds(keep, h), :] + recvs1[s][...]
            r = exch(sem(s, 1), accs[s], recvs2[s], p2)
            r.start()
            st2[s] = r

        st3 = {}
        for s in ORDER:
            _, p1, _, keep, _ = G[s]
            st2[s].wait_recv()
            out_ref[pl.ds(keep, h), :] = accs[s][...] + recvs2[s][...]
            r = exch(sem(s, 2), out_ref.at[pl.ds(keep, h)],
                     out_ref.at[pl.ds(keep, h)], p1)
            r.start()
            st3[s] = r

        for s in ORDER:
            st3[s].wait_recv()
        for rs in (st1, st2, st3):
            for s in ORDER:
                rs[s].wait_send()

    return pl.pallas_call(
        body,
        out_shape=jax.ShapeDtypeStruct((m, n), x.dtype),
        in_specs=[pl.BlockSpec(memory_space=pltpu.VMEM)],
        out_specs=pl.BlockSpec(memory_space=pltpu.VMEM),
        scratch_shapes=(
            [pltpu.VMEM((h, n), x.dtype) for _ in range(N_STREAMS)]
            + [pltpu.VMEM((h, n), x.dtype) for _ in range(N_STREAMS)]
            + [pltpu.VMEM((h, n), x.dtype) for _ in range(N_STREAMS)]
            + [
                pltpu.SemaphoreType.DMA((3 * N_STREAMS,)),
                pltpu.SemaphoreType.DMA((3 * N_STREAMS,)),
            ]
        ),
        compiler_params=pltpu.CompilerParams(collective_id=0),
    )(x)
